# Initial kernel scaffold; baseline (speedup 1.0000x reference)
import functools

import jax
import jax.numpy as jnp
from jax import lax
from jax.experimental import pallas as pl
from jax.experimental.pallas import tpu as pltpu
from jax.experimental.pallas import tpu_sc as plsc

NUM_ROWS = 1000000
DIM = 32
B_TOTAL = 16384 * 26

NC = 2
NS = 16
NW = NC * NS
BPW = B_TOTAL // NW
CHUNK = 512
NCH = BPW // CHUNK


def _make_kernel():
  mesh = plsc.VectorSubcoreMesh(core_axis_name="c", subcore_axis_name="s")

  @functools.partial(
      pl.kernel,
      mesh=mesh,
      out_type=jax.ShapeDtypeStruct((B_TOTAL, DIM), jnp.float32),
      scratch_types=[
          pltpu.VMEM((CHUNK,), jnp.int32),
          pltpu.VMEM((CHUNK, DIM), jnp.float32),
          pltpu.SemaphoreType.DMA,
      ],
  )
  def gather_kernel(idx_hbm, table_hbm, out_hbm, idx_v, rows_v, sem):
    wid = lax.axis_index("s") * NC + lax.axis_index("c")
    base = wid * BPW

    def step(i, carry):
      off = base + i * CHUNK
      pltpu.sync_copy(idx_hbm.at[pl.ds(off, CHUNK)], idx_v)
      pltpu.async_copy(table_hbm.at[idx_v], rows_v, sem).wait()
      pltpu.sync_copy(rows_v, out_hbm.at[pl.ds(off, CHUNK)])
      return carry

    lax.fori_loop(0, NCH, step, 0)

  return gather_kernel


_gather = _make_kernel()


@jax.jit
def kernel(idx, embeddings):
  idx_flat = idx.reshape(B_TOTAL)
  out = _gather(idx_flat, embeddings)
  return out.reshape(idx.shape + (DIM,))


# speedup vs baseline: 1.5169x; 1.5169x over previous
import functools

import jax
import jax.numpy as jnp
from jax import lax
from jax.experimental import pallas as pl
from jax.experimental.pallas import tpu as pltpu
from jax.experimental.pallas import tpu_sc as plsc

NUM_ROWS = 1000000
DIM = 32
B_TOTAL = 16384 * 26

NC = 2
NS = 16
NW = NC * NS
BPW = B_TOTAL // NW
CHUNK = 512
NCH = BPW // CHUNK


def _make_kernel():
  mesh = plsc.VectorSubcoreMesh(core_axis_name="c", subcore_axis_name="s")

  @functools.partial(
      pl.kernel,
      mesh=mesh,
      compiler_params=pltpu.CompilerParams(use_tc_tiling_on_sc=False),
      out_type=jax.ShapeDtypeStruct((B_TOTAL, DIM), jnp.float32),
      scratch_types=[
          pltpu.VMEM((CHUNK,), jnp.int32),
          pltpu.VMEM((CHUNK, DIM), jnp.float32),
          pltpu.SemaphoreType.DMA,
      ],
  )
  def gather_kernel(idx_hbm, table_hbm, out_hbm, idx_v, rows_v, sem):
    wid = lax.axis_index("s") * NC + lax.axis_index("c")
    base = wid * BPW

    def step(i, carry):
      off = base + i * CHUNK
      pltpu.sync_copy(idx_hbm.at[pl.ds(off, CHUNK)], idx_v)
      pltpu.async_copy(table_hbm.at[idx_v], rows_v, sem).wait()
      pltpu.sync_copy(rows_v, out_hbm.at[pl.ds(off, CHUNK)])
      return carry

    lax.fori_loop(0, NCH, step, 0)

  return gather_kernel


_gather = _make_kernel()


@jax.jit
def kernel(idx, embeddings):
  idx_flat = idx.reshape(B_TOTAL)
  out = _gather(idx_flat, embeddings)
  return out.reshape(idx.shape + (DIM,))


# speedup vs baseline: 1.5773x; 1.0398x over previous
import functools

import jax
import jax.numpy as jnp
from jax import lax
from jax.experimental import pallas as pl
from jax.experimental.pallas import tpu as pltpu
from jax.experimental.pallas import tpu_sc as plsc

NUM_ROWS = 1000000
DIM = 32
B_TOTAL = 16384 * 26

NC = 2
NS = 16
NW = NC * NS
BPW = B_TOTAL // NW
CHUNK = 832
NCH = BPW // CHUNK


def _make_kernel():
  mesh = plsc.VectorSubcoreMesh(core_axis_name="c", subcore_axis_name="s")

  @functools.partial(
      pl.kernel,
      mesh=mesh,
      compiler_params=pltpu.CompilerParams(use_tc_tiling_on_sc=False),
      out_type=jax.ShapeDtypeStruct((B_TOTAL, DIM), jnp.float32),
      scratch_types=[
          pltpu.VMEM((BPW,), jnp.int32),
          pltpu.VMEM((CHUNK, DIM), jnp.float32),
          pltpu.VMEM((CHUNK, DIM), jnp.float32),
          pltpu.SemaphoreType.DMA,
          pltpu.SemaphoreType.DMA,
          pltpu.SemaphoreType.DMA,
          pltpu.SemaphoreType.DMA,
      ],
  )
  def gather_kernel(idx_hbm, table_hbm, out_hbm, idx_all, rows0, rows1,
                    g0, g1, s0, s1):
    wid = lax.axis_index("s") * NC + lax.axis_index("c")
    base = wid * BPW
    rows = (rows0, rows1)
    gsem = (g0, g1)
    ssem = (s0, s1)

    pltpu.sync_copy(idx_hbm.at[pl.ds(base, BPW)], idx_all)

    def start_gather(i):
      b = i % 2
      return pltpu.async_copy(
          table_hbm.at[idx_all.at[pl.ds(i * CHUNK, CHUNK)]], rows[b], gsem[b])

    def start_store(i):
      b = i % 2
      return pltpu.async_copy(
          rows[b], out_hbm.at[pl.ds(base + i * CHUNK, CHUNK)], ssem[b])

    gathers = [None] * NCH
    stores = [None] * NCH
    gathers[0] = start_gather(0)
    for i in range(NCH):
      if i + 1 < NCH:
        if i >= 1:
          stores[i - 1].wait()
        gathers[i + 1] = start_gather(i + 1)
      gathers[i].wait()
      stores[i] = start_store(i)
    stores[NCH - 2].wait()
    stores[NCH - 1].wait()

  return gather_kernel


_gather = _make_kernel()


@jax.jit
def kernel(idx, embeddings):
  idx_flat = idx.reshape(B_TOTAL)
  out = _gather(idx_flat, embeddings)
  return out.reshape(idx.shape + (DIM,))
